# baseline (speedup 1.0000x reference)
import math

import jax
import jax.numpy as jnp
from jax.experimental import pallas as pl

_N = 10000
_E = 320000
_D = 128
_DE = 16
_L = 3
_NG = 64

_DELTA = math.log(33.0)


def _bdot(x, w):
    return jnp.dot(x.astype(jnp.bfloat16), w.astype(jnp.bfloat16),
                   preferred_element_type=jnp.float32)


def _msg_body(x_ref, w_ref, b_ref, o_ref):
    y = jnp.dot(x_ref[...], w_ref[...].astype(jnp.bfloat16),
                preferred_element_type=jnp.float32)
    o_ref[...] = jnp.maximum(y + b_ref[...], 0.0)


def _msg_mm(msg_bf16, w, b, block=8000):
    n, k = msg_bf16.shape
    m = w.shape[1]
    return pl.pallas_call(
        _msg_body,
        grid=(n // block,),
        in_specs=[
            pl.BlockSpec((block, k), lambda i: (i, 0)),
            pl.BlockSpec((k, m), lambda i: (0, 0)),
            pl.BlockSpec((1, m), lambda i: (0, 0)),
        ],
        out_specs=pl.BlockSpec((block, m), lambda i: (i, 0)),
        out_shape=jax.ShapeDtypeStruct((n, m), jnp.float32),
    )(msg_bf16, w, b.reshape(1, m))


def _head_body(g_ref, wq_ref, bq_ref, gq_ref, bqq_ref, wo_ref, bo_ref, o_ref):
    y = _bdot(g_ref[...], wq_ref[...]) + bq_ref[...]
    mu = jnp.mean(y, axis=0, keepdims=True)
    var = jnp.mean((y - mu) ** 2, axis=0, keepdims=True)
    y = jnp.maximum(
        (y - mu) / jnp.sqrt(var + 1e-5) * gq_ref[...] + bqq_ref[...], 0.0)
    o_ref[...] = _bdot(y, wo_ref[...]) + bo_ref[...]


def _head(g, wq, bq, gq, bqq, wo, bo):
    return pl.pallas_call(
        _head_body,
        out_shape=jax.ShapeDtypeStruct((_NG, 1), jnp.float32),
    )(g, wq, bq.reshape(1, _D), gq.reshape(1, _D), bqq.reshape(1, _D), wo,
      bo.reshape(1, 1))


def _bn(h, g, b):
    mu = jnp.mean(h, axis=0)
    var = jnp.mean((h - mu) ** 2, axis=0)
    return (h - mu) / jnp.sqrt(var + 1e-5) * g + b


def kernel(x, edge_index, edge_attr, batch, W0, b0, g0, bb0, We, bee, Wm, bm,
           Wp, bp, Wl, bl, gc, bc, Wq, bq, gq, bqq, Wo, bo):
    src = edge_index[0]
    dst = edge_index[1]

    h = jax.nn.relu(_bn(x @ W0 + b0, g0, bb0))

    ones = jnp.ones((_E,), jnp.float32)
    cnt = jax.ops.segment_sum(ones, dst, num_segments=_N)
    cntc = jnp.maximum(cnt, 1.0)
    dlog = jnp.log(cntc + 1.0)
    amp = (dlog / _DELTA)[:, None]
    att = (_DELTA / jnp.maximum(dlog, 1e-5))[:, None]

    for l in range(_L):
        e = edge_attr @ We[l] + bee[l]
        msg_in = jnp.concatenate([h[dst], h[src], e],
                                 axis=1).astype(jnp.bfloat16)
        m = _msg_mm(msg_in, Wm[l], bm[l])
        s = jax.ops.segment_sum(m, dst, num_segments=_N)
        mean = s / cntc[:, None]
        sq = jax.ops.segment_sum(m * m, dst, num_segments=_N) / cntc[:, None]
        std = jnp.sqrt(jax.nn.relu(sq - mean * mean) + 1e-5)
        mn = jax.ops.segment_min(m, dst, num_segments=_N)
        mx = jax.ops.segment_max(m, dst, num_segments=_N)
        has = (cnt > 0)[:, None]
        mn = jnp.where(has, mn, 0.0)
        mx = jnp.where(has, mx, 0.0)
        agg = jnp.concatenate([mean, mn, mx, std], axis=1)
        agg = jnp.concatenate([agg, agg * amp, agg * att], axis=1)
        out = jnp.concatenate([h, agg], axis=1)
        out = jax.nn.relu(out @ Wp[l] + bp[l])
        out = out @ Wl[l] + bl[l]
        h = jax.nn.relu(_bn(out, gc[l], bc[l]))

    gcnt = jnp.maximum(
        jax.ops.segment_sum(jnp.ones((_N,), jnp.float32), batch,
                            num_segments=_NG), 1.0)
    g = jax.ops.segment_sum(h, batch, num_segments=_NG) / gcnt[:, None]

    return _head(g, Wq, bq, gq, bqq, Wo, bo)


# speedup vs baseline: 1.0396x; 1.0396x over previous
import math

import jax
import jax.numpy as jnp
from jax.experimental import pallas as pl

_N = 10000
_E = 320000
_D = 128
_DE = 16
_L = 3
_NG = 64

_DELTA = math.log(33.0)


def _bdot(x, w):
    return jnp.dot(x.astype(jnp.bfloat16), w.astype(jnp.bfloat16),
                   preferred_element_type=jnp.float32)


def _msg_body(x_ref, w_ref, b_ref, o_ref):
    y = jnp.dot(x_ref[...], w_ref[...].astype(jnp.bfloat16),
                preferred_element_type=jnp.float32)
    o_ref[...] = jnp.maximum(y + b_ref[...], 0.0)


def _msg_mm(msg_bf16, w, b, block=8000):
    n, k = msg_bf16.shape
    m = w.shape[1]
    return pl.pallas_call(
        _msg_body,
        grid=(n // block,),
        in_specs=[
            pl.BlockSpec((block, k), lambda i: (i, 0)),
            pl.BlockSpec((k, m), lambda i: (0, 0)),
            pl.BlockSpec((1, m), lambda i: (0, 0)),
        ],
        out_specs=pl.BlockSpec((block, m), lambda i: (i, 0)),
        out_shape=jax.ShapeDtypeStruct((n, m), jnp.float32),
    )(msg_bf16, w, b.reshape(1, m))


def _head_body(g_ref, wq_ref, bq_ref, gq_ref, bqq_ref, wo_ref, bo_ref, o_ref):
    y = _bdot(g_ref[...], wq_ref[...]) + bq_ref[...]
    mu = jnp.mean(y, axis=0, keepdims=True)
    var = jnp.mean((y - mu) ** 2, axis=0, keepdims=True)
    y = jnp.maximum(
        (y - mu) / jnp.sqrt(var + 1e-5) * gq_ref[...] + bqq_ref[...], 0.0)
    o_ref[...] = _bdot(y, wo_ref[...]) + bo_ref[...]


def _head(g, wq, bq, gq, bqq, wo, bo):
    return pl.pallas_call(
        _head_body,
        out_shape=jax.ShapeDtypeStruct((_NG, 1), jnp.float32),
    )(g, wq, bq.reshape(1, _D), gq.reshape(1, _D), bqq.reshape(1, _D), wo,
      bo.reshape(1, 1))


def _bn(h, g, b):
    mu = jnp.mean(h, axis=0)
    var = jnp.mean((h - mu) ** 2, axis=0)
    return (h - mu) / jnp.sqrt(var + 1e-5) * g + b


def kernel(x, edge_index, edge_attr, batch, W0, b0, g0, bb0, We, bee, Wm, bm,
           Wp, bp, Wl, bl, gc, bc, Wq, bq, gq, bqq, Wo, bo):
    src = edge_index[0]
    dst = edge_index[1]

    h = jax.nn.relu(_bn(x @ W0 + b0, g0, bb0))

    ones = jnp.ones((_E,), jnp.float32)
    cnt = jax.ops.segment_sum(ones, dst, num_segments=_N)
    cntc = jnp.maximum(cnt, 1.0)
    dlog = jnp.log(cntc + 1.0)
    amp = (dlog / _DELTA)[:, None]
    att = (_DELTA / jnp.maximum(dlog, 1e-5))[:, None]

    for l in range(_L):
        e = edge_attr @ We[l] + bee[l]
        msg_in = jnp.concatenate([h[dst], h[src], e],
                                 axis=1).astype(jnp.bfloat16)
        m = _msg_mm(msg_in, Wm[l], bm[l])
        s = jax.ops.segment_sum(m, dst, num_segments=_N)
        mean = s / cntc[:, None]
        sq = jax.ops.segment_sum(m * m, dst, num_segments=_N) / cntc[:, None]
        std = jnp.sqrt(jax.nn.relu(sq - mean * mean) + 1e-5)
        mnmx = jax.ops.segment_min(
            jnp.concatenate([m, -m], axis=1), dst, num_segments=_N)
        mn = mnmx[:, :_D]
        mx = -mnmx[:, _D:]
        has = (cnt > 0)[:, None]
        mn = jnp.where(has, mn, 0.0)
        mx = jnp.where(has, mx, 0.0)
        agg = jnp.concatenate([mean, mn, mx, std], axis=1)
        agg = jnp.concatenate([agg, agg * amp, agg * att], axis=1)
        out = jnp.concatenate([h, agg], axis=1)
        out = jax.nn.relu(out @ Wp[l] + bp[l])
        out = out @ Wl[l] + bl[l]
        h = jax.nn.relu(_bn(out, gc[l], bc[l]))

    gcnt = jnp.maximum(
        jax.ops.segment_sum(jnp.ones((_N,), jnp.float32), batch,
                            num_segments=_NG), 1.0)
    g = jax.ops.segment_sum(h, batch, num_segments=_NG) / gcnt[:, None]

    return _head(g, Wq, bq, gq, bqq, Wo, bo)


# speedup vs baseline: 1.0571x; 1.0168x over previous
import math

import jax
import jax.numpy as jnp
from jax.experimental import pallas as pl

_N = 10000
_E = 320000
_D = 128
_DE = 16
_L = 3
_NG = 64

_DELTA = math.log(33.0)


def _bdot(x, w):
    return jnp.dot(x.astype(jnp.bfloat16), w.astype(jnp.bfloat16),
                   preferred_element_type=jnp.float32)


def _msg_body(x_ref, w_ref, b_ref, o_ref, o2_ref, o3_ref):
    y = jnp.dot(x_ref[...], w_ref[...].astype(jnp.bfloat16),
                preferred_element_type=jnp.float32)
    y = jnp.maximum(y + b_ref[...], 0.0)
    o_ref[...] = y
    o2_ref[...] = y * y
    o3_ref[...] = jnp.concatenate([y, -y], axis=1)


def _msg_mm(msg_bf16, w, b, block=8000):
    n, k = msg_bf16.shape
    m = w.shape[1]
    return pl.pallas_call(
        _msg_body,
        grid=(n // block,),
        in_specs=[
            pl.BlockSpec((block, k), lambda i: (i, 0)),
            pl.BlockSpec((k, m), lambda i: (0, 0)),
            pl.BlockSpec((1, m), lambda i: (0, 0)),
        ],
        out_specs=[
            pl.BlockSpec((block, m), lambda i: (i, 0)),
            pl.BlockSpec((block, m), lambda i: (i, 0)),
            pl.BlockSpec((block, 2 * m), lambda i: (i, 0)),
        ],
        out_shape=[
            jax.ShapeDtypeStruct((n, m), jnp.float32),
            jax.ShapeDtypeStruct((n, m), jnp.float32),
            jax.ShapeDtypeStruct((n, 2 * m), jnp.float32),
        ],
    )(msg_bf16, w, b.reshape(1, m))


def _head_body(g_ref, wq_ref, bq_ref, gq_ref, bqq_ref, wo_ref, bo_ref, o_ref):
    y = _bdot(g_ref[...], wq_ref[...]) + bq_ref[...]
    mu = jnp.mean(y, axis=0, keepdims=True)
    var = jnp.mean((y - mu) ** 2, axis=0, keepdims=True)
    y = jnp.maximum(
        (y - mu) / jnp.sqrt(var + 1e-5) * gq_ref[...] + bqq_ref[...], 0.0)
    o_ref[...] = _bdot(y, wo_ref[...]) + bo_ref[...]


def _head(g, wq, bq, gq, bqq, wo, bo):
    return pl.pallas_call(
        _head_body,
        out_shape=jax.ShapeDtypeStruct((_NG, 1), jnp.float32),
    )(g, wq, bq.reshape(1, _D), gq.reshape(1, _D), bqq.reshape(1, _D), wo,
      bo.reshape(1, 1))


def _bn(h, g, b):
    mu = jnp.mean(h, axis=0)
    var = jnp.mean((h - mu) ** 2, axis=0)
    return (h - mu) / jnp.sqrt(var + 1e-5) * g + b


def kernel(x, edge_index, edge_attr, batch, W0, b0, g0, bb0, We, bee, Wm, bm,
           Wp, bp, Wl, bl, gc, bc, Wq, bq, gq, bqq, Wo, bo):
    src = edge_index[0]
    dst = edge_index[1]

    h = jax.nn.relu(_bn(x @ W0 + b0, g0, bb0))

    ones = jnp.ones((_E,), jnp.float32)
    cnt = jax.ops.segment_sum(ones, dst, num_segments=_N)
    cntc = jnp.maximum(cnt, 1.0)
    dlog = jnp.log(cntc + 1.0)
    amp = (dlog / _DELTA)[:, None]
    att = (_DELTA / jnp.maximum(dlog, 1e-5))[:, None]

    for l in range(_L):
        e = edge_attr @ We[l] + bee[l]
        msg_in = jnp.concatenate([h[dst], h[src], e],
                                 axis=1).astype(jnp.bfloat16)
        m, m2, mpm = _msg_mm(msg_in, Wm[l], bm[l])
        s = jax.ops.segment_sum(m, dst, num_segments=_N)
        mean = s / cntc[:, None]
        sq = jax.ops.segment_sum(m2, dst, num_segments=_N) / cntc[:, None]
        std = jnp.sqrt(jax.nn.relu(sq - mean * mean) + 1e-5)
        mnmx = jax.ops.segment_min(mpm, dst, num_segments=_N)
        mn = mnmx[:, :_D]
        mx = -mnmx[:, _D:]
        has = (cnt > 0)[:, None]
        mn = jnp.where(has, mn, 0.0)
        mx = jnp.where(has, mx, 0.0)
        agg = jnp.concatenate([mean, mn, mx, std], axis=1)
        agg = jnp.concatenate([agg, agg * amp, agg * att], axis=1)
        out = jnp.concatenate([h, agg], axis=1)
        out = jax.nn.relu(out @ Wp[l] + bp[l])
        out = out @ Wl[l] + bl[l]
        h = jax.nn.relu(_bn(out, gc[l], bc[l]))

    gcnt = jnp.maximum(
        jax.ops.segment_sum(jnp.ones((_N,), jnp.float32), batch,
                            num_segments=_NG), 1.0)
    g = jax.ops.segment_sum(h, batch, num_segments=_NG) / gcnt[:, None]

    return _head(g, Wq, bq, gq, bqq, Wo, bo)
